# baseline (speedup 1.0000x reference)
import functools

import jax
import jax.numpy as jnp
import numpy as np
from jax import lax
from jax.experimental import pallas as pl
from jax.experimental.pallas import tpu as pltpu
from jax.experimental.pallas import tpu_sc as plsc

_N_OBS = 1000000
_BS = 16384

_NC = 2
_NS = 16
_NW = _NC * _NS
_ROWS_PER_W = _BS // _NW
_CHUNK = 128
_P_PER_W = _ROWS_PER_W * 4
_V_PER_W = _ROWS_PER_W * 8
_P_CHUNKS = _P_PER_W // _CHUNK
_V_CHUNKS = _V_PER_W // _CHUNK


def _compute_batch_indices() -> np.ndarray:
    key = jax.random.key(42)
    key, _ = jax.random.split(key)
    key, subkey = jax.random.split(key)
    perm = jax.random.choice(
        subkey, jnp.arange(_N_OBS), shape=(_N_OBS,), replace=False
    )
    return np.asarray(perm[:_BS], dtype=np.int32)


def _batch_indices() -> np.ndarray:
    try:
        cpu = jax.local_devices(backend="cpu")[0]
    except RuntimeError:
        return _compute_batch_indices()
    with jax.default_device(cpu):
        return _compute_batch_indices()


_IDX = _batch_indices().astype(np.int64)
_PPOS = (np.repeat(_IDX * 4, 4) + np.tile(np.arange(4), _BS)).astype(
    np.int32).reshape(_NW, _P_CHUNKS, _CHUNK)
_VPOS = (np.repeat(_IDX * 8, 8) + np.tile(np.arange(8), _BS)).astype(
    np.int32).reshape(_NW, _V_CHUNKS, _CHUNK)


def _gather_body(pinn_hbm, vals_hbm, ppos_hbm, vpos_hbm,
                 out_pinn, out_vals,
                 ppos_v, vpos_v, opinn_v, ovals_v, sem):
    wid = lax.axis_index("s") * _NC + lax.axis_index("c")
    pltpu.sync_copy(ppos_hbm.at[wid], ppos_v)
    pltpu.sync_copy(vpos_hbm.at[wid], vpos_v)
    copies = []
    for k in range(_P_CHUNKS):
        copies.append(pltpu.async_copy(
            pinn_hbm.at[ppos_v.at[k]], opinn_v.at[pl.ds(k * _CHUNK, _CHUNK)],
            sem))
    for k in range(_V_CHUNKS):
        copies.append(pltpu.async_copy(
            vals_hbm.at[vpos_v.at[k]], ovals_v.at[pl.ds(k * _CHUNK, _CHUNK)],
            sem))
    for cp in copies:
        cp.wait()
    pltpu.sync_copy(opinn_v, out_pinn.at[pl.ds(wid * _P_PER_W, _P_PER_W)])
    pltpu.sync_copy(ovals_v, out_vals.at[pl.ds(wid * _V_PER_W, _V_PER_W)])


@functools.cache
def _sc_gather():
    mesh = plsc.VectorSubcoreMesh(core_axis_name="c", subcore_axis_name="s")
    return pl.kernel(
        _gather_body,
        mesh=mesh,
        out_type=(
            jax.ShapeDtypeStruct((_BS * 4,), jnp.float32),
            jax.ShapeDtypeStruct((_BS * 8,), jnp.float32),
        ),
        scratch_types=[
            pltpu.VMEM((_P_CHUNKS, _CHUNK), jnp.int32),
            pltpu.VMEM((_V_CHUNKS, _CHUNK), jnp.int32),
            pltpu.VMEM((_P_PER_W,), jnp.float32),
            pltpu.VMEM((_V_PER_W,), jnp.float32),
            pltpu.SemaphoreType.DMA,
        ],
        compiler_params=pltpu.CompilerParams(
            use_tc_tiling_on_sc=False, needs_layout_passes=False),
    )


def kernel(observed_pinn_in, observed_values):
    pinn_flat, vals_flat = _sc_gather()(
        observed_pinn_in.reshape(_N_OBS * 4),
        observed_values.reshape(_N_OBS * 8),
        jnp.asarray(_PPOS), jnp.asarray(_VPOS))
    return (pinn_flat.reshape(_BS, 4), vals_flat.reshape(_BS, 8))


# speedup vs baseline: 2.5400x; 2.5400x over previous
import functools

import jax
import jax.numpy as jnp
import numpy as np
from jax import lax
from jax.experimental import pallas as pl
from jax.experimental.pallas import tpu as pltpu
from jax.experimental.pallas import tpu_sc as plsc

_N_OBS = 1000000
_BS = 16384

_NC = 2
_NS = 16
_NW = _NC * _NS
_ROWS_PER_W = _BS // _NW
_CHUNK = 128
_P_PER_W = _ROWS_PER_W * 4
_V_PER_W = _ROWS_PER_W * 8
_P_CHUNKS = _P_PER_W // _CHUNK
_V_CHUNKS = _V_PER_W // _CHUNK


def _compute_batch_indices() -> np.ndarray:
    key = jax.random.key(42)
    key, _ = jax.random.split(key)
    key, subkey = jax.random.split(key)
    perm = jax.random.choice(
        subkey, jnp.arange(_N_OBS), shape=(_N_OBS,), replace=False
    )
    return np.asarray(perm[:_BS], dtype=np.int32)


def _batch_indices() -> np.ndarray:
    try:
        cpu = jax.local_devices(backend="cpu")[0]
    except RuntimeError:
        return _compute_batch_indices()
    with jax.default_device(cpu):
        return _compute_batch_indices()


_IDX = _batch_indices().astype(np.int64)


def _positions(ncols: int) -> np.ndarray:
    rows = _IDX.reshape(_NW, _ROWS_PER_W)
    cols = np.arange(ncols, dtype=np.int64)
    pos = cols[None, :, None] * _N_OBS + rows[:, None, :]
    return pos.astype(np.int32).reshape(_NW, ncols * _ROWS_PER_W // _CHUNK,
                                        _CHUNK)


_PPOS = _positions(4)
_VPOS = _positions(8)


def _gather_body(pinn_hbm, vals_hbm, ppos_hbm, vpos_hbm,
                 out_pinn, out_vals,
                 ppos_v, vpos_v, opinn_v, ovals_v, sem):
    wid = lax.axis_index("s") * _NC + lax.axis_index("c")
    pltpu.sync_copy(ppos_hbm.at[wid], ppos_v)
    pltpu.sync_copy(vpos_hbm.at[wid], vpos_v)
    copies = []
    for k in range(_P_CHUNKS):
        copies.append(pltpu.async_copy(
            pinn_hbm.at[ppos_v.at[k]], opinn_v.at[pl.ds(k * _CHUNK, _CHUNK)],
            sem))
    for k in range(_V_CHUNKS):
        copies.append(pltpu.async_copy(
            vals_hbm.at[vpos_v.at[k]], ovals_v.at[pl.ds(k * _CHUNK, _CHUNK)],
            sem))
    for cp in copies:
        cp.wait()
    base = wid * _ROWS_PER_W
    for c in range(4):
        pltpu.sync_copy(
            opinn_v.at[pl.ds(c * _ROWS_PER_W, _ROWS_PER_W)],
            out_pinn.at[c, pl.ds(base, _ROWS_PER_W)])
    for c in range(8):
        pltpu.sync_copy(
            ovals_v.at[pl.ds(c * _ROWS_PER_W, _ROWS_PER_W)],
            out_vals.at[c, pl.ds(base, _ROWS_PER_W)])


@functools.cache
def _sc_gather():
    mesh = plsc.VectorSubcoreMesh(core_axis_name="c", subcore_axis_name="s")
    return pl.kernel(
        _gather_body,
        mesh=mesh,
        out_type=(
            jax.ShapeDtypeStruct((4, _BS), jnp.float32),
            jax.ShapeDtypeStruct((8, _BS), jnp.float32),
        ),
        scratch_types=[
            pltpu.VMEM((_P_CHUNKS, _CHUNK), jnp.int32),
            pltpu.VMEM((_V_CHUNKS, _CHUNK), jnp.int32),
            pltpu.VMEM((_P_PER_W,), jnp.float32),
            pltpu.VMEM((_V_PER_W,), jnp.float32),
            pltpu.SemaphoreType.DMA,
        ],
        compiler_params=pltpu.CompilerParams(
            use_tc_tiling_on_sc=False, needs_layout_passes=False),
    )


def kernel(observed_pinn_in, observed_values):
    pinn_t, vals_t = _sc_gather()(
        observed_pinn_in.T.reshape(_N_OBS * 4),
        observed_values.T.reshape(_N_OBS * 8),
        jnp.asarray(_PPOS), jnp.asarray(_VPOS))
    return (pinn_t.T, vals_t.T)


# speedup vs baseline: 15.5930x; 6.1391x over previous
import functools

import jax
import jax.numpy as jnp
import numpy as np
from jax import lax
from jax.experimental import pallas as pl
from jax.experimental.pallas import tpu as pltpu
from jax.experimental.pallas import tpu_sc as plsc

_N_OBS = 1000000
_BS = 16384

_NC = 2
_NS = 16
_NW = _NC * _NS
_ROWS_PER_W = _BS // _NW
_CHUNK = 128
_KL = _ROWS_PER_W // _CHUNK

_STRIPE = 128
_NFULL = (_N_OBS // _STRIPE) * _STRIPE
_NSTR = _NFULL // _STRIPE
_NTAIL = _N_OBS - _NFULL


def _compute_batch_indices() -> np.ndarray:
    key = jax.random.key(42)
    key, _ = jax.random.split(key)
    key, subkey = jax.random.split(key)
    perm = jax.random.choice(
        subkey, jnp.arange(_N_OBS), shape=(_N_OBS,), replace=False
    )
    return np.asarray(perm[:_BS], dtype=np.int32)


def _batch_indices() -> np.ndarray:
    try:
        cpu = jax.local_devices(backend="cpu")[0]
    except RuntimeError:
        return _compute_batch_indices()
    with jax.default_device(cpu):
        return _compute_batch_indices()


_IDX = _batch_indices().astype(np.int64)


def _stripe_positions(ncols: int) -> np.ndarray:
    r = _IDX.reshape(_NW, _KL, 1, _CHUNK)
    c = np.arange(ncols, dtype=np.int64).reshape(1, 1, ncols, 1)
    pos = (r // _STRIPE) * (ncols * _STRIPE) + c * _STRIPE + (r % _STRIPE)
    pos = np.where(r >= _NFULL, 0, pos)
    return pos.astype(np.int32).reshape(_NW, _KL * ncols, _CHUNK)


_PPOS = _stripe_positions(4)
_VPOS = _stripe_positions(8)

_TAIL_SLOTS = np.nonzero(_IDX >= _NFULL)[0]
assert len(_TAIL_SLOTS) <= 16, "masked single-chunk fix-up assumed"
_FIX_W = int(_TAIL_SLOTS[0] // _ROWS_PER_W) if len(_TAIL_SLOTS) else -1
assert all(int(s // _ROWS_PER_W) == _FIX_W for s in _TAIL_SLOTS)


def _fix_vectors(ncols: int) -> tuple[np.ndarray, np.ndarray, int]:
    src = np.zeros(16, np.int32)
    dst = np.zeros(16, np.int32)
    n = 0
    for s in _TAIL_SLOTS:
        t = int(_IDX[s] - _NFULL)
        lr = int(s % _ROWS_PER_W)
        kl, j = lr // _CHUNK, lr % _CHUNK
        for c in range(ncols):
            src[n] = t * ncols + c
            dst[n] = kl * (ncols * _CHUNK) + c * _CHUNK + j
            n += 1
    return src, dst, n


_PSRC, _PDST, _PN = _fix_vectors(4)
_VSRC, _VDST, _VN = _fix_vectors(8)
_FIX = np.concatenate([_PSRC, _PDST, _VSRC, _VDST])


def _gather_body(pinn_hbm, vals_hbm, tailp_hbm, tailv_hbm,
                 ppos_hbm, vpos_hbm, fix_hbm,
                 out_pinn, out_vals,
                 ppos_v, vpos_v, opinn_v, ovals_v,
                 tailp_v, tailv_v, fix_v, sem):
    wid = lax.axis_index("s") * _NC + lax.axis_index("c")
    pltpu.sync_copy(ppos_hbm.at[wid], ppos_v)
    pltpu.sync_copy(vpos_hbm.at[wid], vpos_v)
    copies = []
    for k in range(_KL * 4):
        copies.append(pltpu.async_copy(
            pinn_hbm.at[ppos_v.at[k]], opinn_v.at[pl.ds(k * _CHUNK, _CHUNK)],
            sem))
    for k in range(_KL * 8):
        copies.append(pltpu.async_copy(
            vals_hbm.at[vpos_v.at[k]], ovals_v.at[pl.ds(k * _CHUNK, _CHUNK)],
            sem))
    for cp in copies:
        cp.wait()
    if _FIX_W >= 0:
        @pl.when(wid == _FIX_W)
        def _fixup():
            pltpu.sync_copy(tailp_hbm, tailp_v)
            pltpu.sync_copy(tailv_hbm, tailv_v)
            pltpu.sync_copy(fix_hbm, fix_v)
            lane = lax.iota(jnp.int32, 16)
            psrc = fix_v[pl.ds(0, 16)]
            pdst = fix_v[pl.ds(16, 16)]
            vsrc = fix_v[pl.ds(32, 16)]
            vdst = fix_v[pl.ds(48, 16)]
            plsc.store_scatter(opinn_v, [pdst],
                               plsc.load_gather(tailp_v, [psrc]),
                               mask=lane < _PN)
            plsc.store_scatter(ovals_v, [vdst],
                               plsc.load_gather(tailv_v, [vsrc]),
                               mask=lane < _VN)
    pltpu.sync_copy(opinn_v,
                    out_pinn.at[pl.ds(wid * _KL * 4 * _CHUNK, _KL * 4 * _CHUNK)])
    pltpu.sync_copy(ovals_v,
                    out_vals.at[pl.ds(wid * _KL * 8 * _CHUNK, _KL * 8 * _CHUNK)])


@functools.cache
def _sc_gather():
    mesh = plsc.VectorSubcoreMesh(core_axis_name="c", subcore_axis_name="s")
    return pl.kernel(
        _gather_body,
        mesh=mesh,
        out_type=(
            jax.ShapeDtypeStruct((_BS * 4,), jnp.float32),
            jax.ShapeDtypeStruct((_BS * 8,), jnp.float32),
        ),
        scratch_types=[
            pltpu.VMEM((_KL * 4, _CHUNK), jnp.int32),
            pltpu.VMEM((_KL * 8, _CHUNK), jnp.int32),
            pltpu.VMEM((_KL * 4 * _CHUNK,), jnp.float32),
            pltpu.VMEM((_KL * 8 * _CHUNK,), jnp.float32),
            pltpu.VMEM((_NTAIL * 4,), jnp.float32),
            pltpu.VMEM((_NTAIL * 8,), jnp.float32),
            pltpu.VMEM((64,), jnp.int32),
            pltpu.SemaphoreType.DMA,
        ],
        compiler_params=pltpu.CompilerParams(
            use_tc_tiling_on_sc=False, needs_layout_passes=False),
    )


def kernel(observed_pinn_in, observed_values):
    pinn_s = observed_pinn_in[:_NFULL].reshape(
        _NSTR, _STRIPE, 4).transpose(0, 2, 1).reshape(-1)
    vals_s = observed_values[:_NFULL].reshape(
        _NSTR, _STRIPE, 8).transpose(0, 2, 1).reshape(-1)
    tailp = observed_pinn_in[_NFULL:].reshape(-1)
    tailv = observed_values[_NFULL:].reshape(-1)
    out_p, out_v = _sc_gather()(
        pinn_s, vals_s, tailp, tailv,
        jnp.asarray(_PPOS), jnp.asarray(_VPOS), jnp.asarray(_FIX))
    pinn_b = out_p.reshape(_BS // _STRIPE, 4, _STRIPE).transpose(
        0, 2, 1).reshape(_BS, 4)
    vals_b = out_v.reshape(_BS // _STRIPE, 8, _STRIPE).transpose(
        0, 2, 1).reshape(_BS, 8)
    return (pinn_b, vals_b)


# speedup vs baseline: 22.2428x; 1.4265x over previous
import functools

import jax
import jax.numpy as jnp
import numpy as np
from jax import lax
from jax.experimental import pallas as pl
from jax.experimental.pallas import tpu as pltpu
from jax.experimental.pallas import tpu_sc as plsc

_N_OBS = 1000000
_BS = 16384

_NC = 2
_NS = 16
_NW = _NC * _NS
_ROWS_PER_W = _BS // _NW
_CHUNK = 128
_KL = _ROWS_PER_W // _CHUNK

_STRIPE = 128
_NFULL = (_N_OBS // _STRIPE) * _STRIPE
_NSTR = _NFULL // _STRIPE
_NTAIL = _N_OBS - _NFULL


def _compute_batch_indices() -> np.ndarray:
    key = jax.random.key(42)
    key, _ = jax.random.split(key)
    key, subkey = jax.random.split(key)
    perm = jax.random.choice(
        subkey, jnp.arange(_N_OBS), shape=(_N_OBS,), replace=False
    )
    return np.asarray(perm[:_BS], dtype=np.int32)


def _batch_indices() -> np.ndarray:
    try:
        cpu = jax.local_devices(backend="cpu")[0]
    except RuntimeError:
        return _compute_batch_indices()
    with jax.default_device(cpu):
        return _compute_batch_indices()


_IDX = _batch_indices().astype(np.int64)


def _stripe_positions(ncols: int, ncols_phys: int) -> np.ndarray:
    r = _IDX.reshape(_NW, _KL, 1, _CHUNK)
    c = np.arange(ncols, dtype=np.int64).reshape(1, 1, ncols, 1)
    pos = (r // _STRIPE) * (ncols_phys * _STRIPE) + c * _STRIPE + (r % _STRIPE)
    pos = np.where(r >= _NFULL, 0, pos)
    return pos.astype(np.int32).reshape(_NW, _KL * ncols, _CHUNK)


_PPOS = _stripe_positions(4, 8)
_VPOS = _stripe_positions(8, 8)

_TAIL_SLOTS = np.nonzero(_IDX >= _NFULL)[0]
assert len(_TAIL_SLOTS) <= 16, "masked single-chunk fix-up assumed"
_FIX_W = int(_TAIL_SLOTS[0] // _ROWS_PER_W) if len(_TAIL_SLOTS) else -1
assert all(int(s // _ROWS_PER_W) == _FIX_W for s in _TAIL_SLOTS)


def _fix_vectors(ncols: int) -> tuple[np.ndarray, np.ndarray, int]:
    src = np.zeros(16, np.int32)
    dst = np.zeros(16, np.int32)
    n = 0
    for s in _TAIL_SLOTS:
        t = int(_IDX[s] - _NFULL)
        lr = int(s % _ROWS_PER_W)
        kl, j = lr // _CHUNK, lr % _CHUNK
        for c in range(ncols):
            src[n] = t * ncols + c
            dst[n] = kl * (ncols * _CHUNK) + c * _CHUNK + j
            n += 1
    return src, dst, n


_PSRC, _PDST, _PN = _fix_vectors(4)
_VSRC, _VDST, _VN = _fix_vectors(8)
_FIX = np.concatenate([_PSRC, _PDST, _VSRC, _VDST])


def _gather_body(pinn_hbm, vals_hbm, tailp_hbm, tailv_hbm,
                 ppos_hbm, vpos_hbm, fix_hbm,
                 out_pinn, out_vals,
                 ppos_v, vpos_v, opinn_v, ovals_v,
                 tailp_v, tailv_v, fix_v, sem):
    wid = lax.axis_index("s") * _NC + lax.axis_index("c")
    pltpu.sync_copy(ppos_hbm.at[wid], ppos_v)
    pltpu.sync_copy(vpos_hbm.at[wid], vpos_v)
    copies = []
    for k in range(_KL * 4):
        copies.append(pltpu.async_copy(
            pinn_hbm.at[ppos_v.at[k]], opinn_v.at[pl.ds(k * _CHUNK, _CHUNK)],
            sem))
    for k in range(_KL * 8):
        copies.append(pltpu.async_copy(
            vals_hbm.at[vpos_v.at[k]], ovals_v.at[pl.ds(k * _CHUNK, _CHUNK)],
            sem))
    for cp in copies:
        cp.wait()
    if _FIX_W >= 0:
        @pl.when(wid == _FIX_W)
        def _fixup():
            pltpu.sync_copy(tailp_hbm, tailp_v)
            pltpu.sync_copy(tailv_hbm, tailv_v)
            pltpu.sync_copy(fix_hbm, fix_v)
            lane = lax.iota(jnp.int32, 16)
            psrc = fix_v[pl.ds(0, 16)]
            pdst = fix_v[pl.ds(16, 16)]
            vsrc = fix_v[pl.ds(32, 16)]
            vdst = fix_v[pl.ds(48, 16)]
            plsc.store_scatter(opinn_v, [pdst],
                               plsc.load_gather(tailp_v, [psrc]),
                               mask=lane < _PN)
            plsc.store_scatter(ovals_v, [vdst],
                               plsc.load_gather(tailv_v, [vsrc]),
                               mask=lane < _VN)
    pltpu.sync_copy(opinn_v,
                    out_pinn.at[pl.ds(wid * _KL * 4 * _CHUNK, _KL * 4 * _CHUNK)])
    pltpu.sync_copy(ovals_v,
                    out_vals.at[pl.ds(wid * _KL * 8 * _CHUNK, _KL * 8 * _CHUNK)])


@functools.cache
def _sc_gather():
    mesh = plsc.VectorSubcoreMesh(core_axis_name="c", subcore_axis_name="s")
    return pl.kernel(
        _gather_body,
        mesh=mesh,
        out_type=(
            jax.ShapeDtypeStruct((_BS * 4,), jnp.float32),
            jax.ShapeDtypeStruct((_BS * 8,), jnp.float32),
        ),
        name="minibatch_gather",
        scratch_types=[
            pltpu.VMEM((_KL * 4, _CHUNK), jnp.int32),
            pltpu.VMEM((_KL * 8, _CHUNK), jnp.int32),
            pltpu.VMEM((_KL * 4 * _CHUNK,), jnp.float32),
            pltpu.VMEM((_KL * 8 * _CHUNK,), jnp.float32),
            pltpu.VMEM((_NTAIL * 4,), jnp.float32),
            pltpu.VMEM((_NTAIL * 8,), jnp.float32),
            pltpu.VMEM((64,), jnp.int32),
            pltpu.SemaphoreType.DMA,
        ],
        compiler_params=pltpu.CompilerParams(
            use_tc_tiling_on_sc=False, needs_layout_passes=False),
    )


def kernel(observed_pinn_in, observed_values):
    pinn8 = jnp.pad(observed_pinn_in[:_NFULL], ((0, 0), (0, 4)))
    pinn_s = pinn8.reshape(
        _NSTR, _STRIPE, 8).transpose(0, 2, 1).reshape(-1)
    vals_s = observed_values[:_NFULL].reshape(
        _NSTR, _STRIPE, 8).transpose(0, 2, 1).reshape(-1)
    tailp = observed_pinn_in[_NFULL:].reshape(-1)
    tailv = observed_values[_NFULL:].reshape(-1)
    out_p, out_v = _sc_gather()(
        pinn_s, vals_s, tailp, tailv,
        jnp.asarray(_PPOS), jnp.asarray(_VPOS), jnp.asarray(_FIX))
    pinn_b = out_p.reshape(_BS // _STRIPE, 4, _STRIPE).transpose(
        0, 2, 1).reshape(_BS, 4)
    vals_b = out_v.reshape(_BS // _STRIPE, 8, _STRIPE).transpose(
        0, 2, 1).reshape(_BS, 8)
    return (pinn_b, vals_b)


# speedup vs baseline: 22.3212x; 1.0035x over previous
import functools

import jax
import jax.numpy as jnp
import numpy as np
from jax import lax
from jax.experimental import pallas as pl
from jax.experimental.pallas import tpu as pltpu
from jax.experimental.pallas import tpu_sc as plsc

_N_OBS = 1000000
_BS = 16384

_NC = 2
_NS = 16
_NW = _NC * _NS
_ROWS_PER_W = _BS // _NW
_CHUNK = 128
_KL = _ROWS_PER_W // _CHUNK

_STRIPE = 128
_NFULL = (_N_OBS // _STRIPE) * _STRIPE
_NSTR = _NFULL // _STRIPE
_NTAIL = _N_OBS - _NFULL


def _compute_batch_indices() -> np.ndarray:
    key = jax.random.key(42)
    key, _ = jax.random.split(key)
    key, subkey = jax.random.split(key)
    perm = jax.random.choice(
        subkey, jnp.arange(_N_OBS), shape=(_N_OBS,), replace=False
    )
    return np.asarray(perm[:_BS], dtype=np.int32)


def _batch_indices() -> np.ndarray:
    try:
        cpu = jax.local_devices(backend="cpu")[0]
    except RuntimeError:
        return _compute_batch_indices()
    with jax.default_device(cpu):
        return _compute_batch_indices()


_IDX = _batch_indices().astype(np.int64)


def _stripe_positions(ncols: int, ncols_phys: int) -> np.ndarray:
    r = _IDX.reshape(_NW, _KL, 1, _CHUNK)
    c = np.arange(ncols, dtype=np.int64).reshape(1, 1, ncols, 1)
    pos = (r // _STRIPE) * (ncols_phys * _STRIPE) + c * _STRIPE + (r % _STRIPE)
    pos = np.where(r >= _NFULL, 0, pos)
    return pos.astype(np.int32).reshape(_NW, _KL * ncols, _CHUNK)


_PPOS = _stripe_positions(4, 8)
_VPOS = _stripe_positions(8, 8)

_TAIL_SLOTS = np.nonzero(_IDX >= _NFULL)[0]
assert len(_TAIL_SLOTS) <= 16, "masked single-chunk fix-up assumed"
_FIX_W = int(_TAIL_SLOTS[0] // _ROWS_PER_W) if len(_TAIL_SLOTS) else -1
assert all(int(s // _ROWS_PER_W) == _FIX_W for s in _TAIL_SLOTS)


def _fix_vectors(ncols: int) -> tuple[np.ndarray, np.ndarray, np.ndarray, int]:
    src = np.zeros(16, np.int32)
    dst0 = np.zeros(16, np.int32)
    dst1 = np.zeros(16, np.int32)
    n = 0
    for s in _TAIL_SLOTS:
        t = int(_IDX[s] - _NFULL)
        lr = int(s % _ROWS_PER_W)
        kl, j = lr // _CHUNK, lr % _CHUNK
        for c in range(ncols):
            src[n] = t * ncols + c
            d = kl * (ncols * _CHUNK) + c * _CHUNK + j
            dst0[n], dst1[n] = d // _CHUNK, d % _CHUNK
            n += 1
    return src, dst0, dst1, n


_PSRC, _PDST0, _PDST1, _PN = _fix_vectors(4)
_VSRC, _VDST0, _VDST1, _VN = _fix_vectors(8)
_FIX = np.concatenate([_PSRC, _PDST0, _PDST1, _VSRC, _VDST0, _VDST1])


def _gather_body(pinn_hbm, vals_hbm, tailp_hbm, tailv_hbm,
                 ppos_hbm, vpos_hbm, fix_hbm,
                 out_pinn, out_vals,
                 ppos_v, vpos_v, opinn_v, ovals_v,
                 tailp_v, tailv_v, fix_v, sem):
    wid = lax.axis_index("s") * _NC + lax.axis_index("c")
    pltpu.sync_copy(ppos_hbm.at[wid], ppos_v)
    pltpu.sync_copy(vpos_hbm.at[wid], vpos_v)
    copies = []
    for k in range(_KL * 4):
        copies.append(pltpu.async_copy(
            pinn_hbm.at[ppos_v.at[k]], opinn_v.at[k], sem))
    for k in range(_KL * 8):
        copies.append(pltpu.async_copy(
            vals_hbm.at[vpos_v.at[k]], ovals_v.at[k], sem))
    for cp in copies:
        cp.wait()
    if _FIX_W >= 0:
        @pl.when(wid == _FIX_W)
        def _fixup():
            pltpu.sync_copy(tailp_hbm, tailp_v)
            pltpu.sync_copy(tailv_hbm, tailv_v)
            pltpu.sync_copy(fix_hbm, fix_v)
            lane = lax.iota(jnp.int32, 16)
            plsc.store_scatter(
                opinn_v,
                [fix_v[pl.ds(16, 16)], fix_v[pl.ds(32, 16)]],
                plsc.load_gather(tailp_v, [fix_v[pl.ds(0, 16)]]),
                mask=lane < _PN)
            plsc.store_scatter(
                ovals_v,
                [fix_v[pl.ds(64, 16)], fix_v[pl.ds(80, 16)]],
                plsc.load_gather(tailv_v, [fix_v[pl.ds(48, 16)]]),
                mask=lane < _VN)
    pltpu.sync_copy(opinn_v, out_pinn.at[pl.ds(wid * _KL * 4, _KL * 4)])
    pltpu.sync_copy(ovals_v, out_vals.at[pl.ds(wid * _KL * 8, _KL * 8)])


@functools.cache
def _sc_gather():
    mesh = plsc.VectorSubcoreMesh(core_axis_name="c", subcore_axis_name="s")
    return pl.kernel(
        _gather_body,
        mesh=mesh,
        out_type=(
            jax.ShapeDtypeStruct((_BS * 4 // _CHUNK, _CHUNK), jnp.float32),
            jax.ShapeDtypeStruct((_BS * 8 // _CHUNK, _CHUNK), jnp.float32),
        ),
        name="minibatch_gather",
        scratch_types=[
            pltpu.VMEM((_KL * 4, _CHUNK), jnp.int32),
            pltpu.VMEM((_KL * 8, _CHUNK), jnp.int32),
            pltpu.VMEM((_KL * 4, _CHUNK), jnp.float32),
            pltpu.VMEM((_KL * 8, _CHUNK), jnp.float32),
            pltpu.VMEM((_NTAIL * 4,), jnp.float32),
            pltpu.VMEM((_NTAIL * 8,), jnp.float32),
            pltpu.VMEM((96,), jnp.int32),
            pltpu.SemaphoreType.DMA,
        ],
        compiler_params=pltpu.CompilerParams(
            use_tc_tiling_on_sc=False, needs_layout_passes=False),
    )


def kernel(observed_pinn_in, observed_values):
    pinn8 = jnp.pad(observed_pinn_in[:_NFULL], ((0, 0), (0, 4)))
    pinn_s = pinn8.reshape(
        _NSTR, _STRIPE, 8).transpose(0, 2, 1).reshape(-1)
    vals_s = observed_values[:_NFULL].reshape(
        _NSTR, _STRIPE, 8).transpose(0, 2, 1).reshape(-1)
    tailp = observed_pinn_in[_NFULL:].reshape(-1)
    tailv = observed_values[_NFULL:].reshape(-1)
    out_p, out_v = _sc_gather()(
        pinn_s, vals_s, tailp, tailv,
        jnp.asarray(_PPOS), jnp.asarray(_VPOS), jnp.asarray(_FIX))
    pinn_b = out_p.reshape(_BS // _STRIPE, 4, _STRIPE).transpose(
        0, 2, 1).reshape(_BS, 4)
    vals_b = out_v.reshape(_BS // _STRIPE, 8, _STRIPE).transpose(
        0, 2, 1).reshape(_BS, 8)
    return (pinn_b, vals_b)


# speedup vs baseline: 22.3612x; 1.0018x over previous
import functools

import jax
import jax.numpy as jnp
import numpy as np
from jax import lax
from jax.experimental import pallas as pl
from jax.experimental.pallas import tpu as pltpu
from jax.experimental.pallas import tpu_sc as plsc

_N_OBS = 1000000
_BS = 16384

_NC = 2
_NS = 16
_NW = _NC * _NS
_ROWS_PER_W = _BS // _NW
_CHUNK = 128
_KL = _ROWS_PER_W // _CHUNK

_STRIPE = 128
_NFULL = (_N_OBS // _STRIPE) * _STRIPE
_NSTR = _NFULL // _STRIPE
_NTAIL = _N_OBS - _NFULL


def _compute_batch_indices() -> np.ndarray:
    key = jax.random.key(42)
    key, _ = jax.random.split(key)
    key, subkey = jax.random.split(key)
    perm = jax.random.choice(
        subkey, jnp.arange(_N_OBS), shape=(_N_OBS,), replace=False
    )
    return np.asarray(perm[:_BS], dtype=np.int32)


def _batch_indices() -> np.ndarray:
    try:
        cpu = jax.local_devices(backend="cpu")[0]
    except RuntimeError:
        return _compute_batch_indices()
    with jax.default_device(cpu):
        return _compute_batch_indices()


_IDX = _batch_indices().astype(np.int64)


def _stripe_positions(ncols: int, ncols_phys: int) -> np.ndarray:
    r = _IDX.reshape(_NW, _KL, 1, _CHUNK)
    c = np.arange(ncols, dtype=np.int64).reshape(1, 1, ncols, 1)
    pos = (r // _STRIPE) * (ncols_phys * _STRIPE) + c * _STRIPE + (r % _STRIPE)
    pos = np.where(r >= _NFULL, 0, pos)
    return pos.astype(np.int32).reshape(_NW, _KL * ncols, _CHUNK)


_PPOS = _stripe_positions(4, 8)
_VPOS = _stripe_positions(8, 8)

_TAIL_SLOTS = np.nonzero(_IDX >= _NFULL)[0]
assert len(_TAIL_SLOTS) <= 16, "masked single-chunk fix-up assumed"
_FIX_W = int(_TAIL_SLOTS[0] // _ROWS_PER_W) if len(_TAIL_SLOTS) else -1
assert all(int(s // _ROWS_PER_W) == _FIX_W for s in _TAIL_SLOTS)


def _fix_vectors(ncols: int) -> tuple[np.ndarray, np.ndarray, np.ndarray, int]:
    src = np.zeros(16, np.int32)
    dst0 = np.zeros(16, np.int32)
    dst1 = np.zeros(16, np.int32)
    n = 0
    for s in _TAIL_SLOTS:
        t = int(_IDX[s] - _NFULL)
        lr = int(s % _ROWS_PER_W)
        kl, j = lr // _CHUNK, lr % _CHUNK
        for c in range(ncols):
            src[n] = t * ncols + c
            d = kl * (ncols * _CHUNK) + c * _CHUNK + j
            dst0[n], dst1[n] = d // _CHUNK, d % _CHUNK
            n += 1
    return src, dst0, dst1, n


_PSRC, _PDST0, _PDST1, _PN = _fix_vectors(4)
_VSRC, _VDST0, _VDST1, _VN = _fix_vectors(8)
_FIX = np.concatenate([_PSRC, _PDST0, _PDST1, _VSRC, _VDST0, _VDST1])


def _gather_body(pinn_hbm, vals_hbm, tailp_hbm, tailv_hbm,
                 ppos_hbm, vpos_hbm, fix_hbm,
                 out_pinn, out_vals,
                 ppos_v, vpos_v, opinn_v, ovals_v,
                 tailp_v, tailv_v, fix_v, sem):
    wid = lax.axis_index("s") * _NC + lax.axis_index("c")
    pltpu.sync_copy(ppos_hbm.at[wid], ppos_v)
    pltpu.sync_copy(vpos_hbm.at[wid], vpos_v)
    copies = []
    for k in range(_KL * 4):
        copies.append(pltpu.async_copy(
            pinn_hbm.at[ppos_v.at[k]], opinn_v.at[k], sem))
    for k in range(_KL * 8):
        copies.append(pltpu.async_copy(
            vals_hbm.at[vpos_v.at[k]], ovals_v.at[k], sem))
    for cp in copies:
        cp.wait()
    if _FIX_W >= 0:
        @pl.when(wid == _FIX_W)
        def _fixup():
            pltpu.sync_copy(tailp_hbm, tailp_v)
            pltpu.sync_copy(tailv_hbm, tailv_v)
            pltpu.sync_copy(fix_hbm, fix_v)
            lane = lax.iota(jnp.int32, 16)
            plsc.store_scatter(
                opinn_v,
                [fix_v[pl.ds(16, 16)], fix_v[pl.ds(32, 16)]],
                plsc.load_gather(tailp_v, [fix_v[pl.ds(0, 16)]]),
                mask=lane < _PN)
            plsc.store_scatter(
                ovals_v,
                [fix_v[pl.ds(64, 16)], fix_v[pl.ds(80, 16)]],
                plsc.load_gather(tailv_v, [fix_v[pl.ds(48, 16)]]),
                mask=lane < _VN)
    pltpu.sync_copy(opinn_v, out_pinn.at[pl.ds(wid * _KL * 4, _KL * 4)])
    pltpu.sync_copy(ovals_v, out_vals.at[pl.ds(wid * _KL * 8, _KL * 8)])


@functools.cache
def _sc_gather():
    mesh = plsc.VectorSubcoreMesh(core_axis_name="c", subcore_axis_name="s")
    return pl.kernel(
        _gather_body,
        mesh=mesh,
        out_type=(
            jax.ShapeDtypeStruct((_BS * 4 // _CHUNK, _CHUNK), jnp.float32),
            jax.ShapeDtypeStruct((_BS * 8 // _CHUNK, _CHUNK), jnp.float32),
        ),
        name="minibatch_gather",
        scratch_types=[
            pltpu.VMEM((_KL * 4, _CHUNK), jnp.int32),
            pltpu.VMEM((_KL * 8, _CHUNK), jnp.int32),
            pltpu.VMEM((_KL * 4, _CHUNK), jnp.float32),
            pltpu.VMEM((_KL * 8, _CHUNK), jnp.float32),
            pltpu.VMEM((_NTAIL * 4,), jnp.float32),
            pltpu.VMEM((_NTAIL * 8,), jnp.float32),
            pltpu.VMEM((96,), jnp.int32),
            pltpu.SemaphoreType.DMA,
        ],
        compiler_params=pltpu.CompilerParams(
            use_tc_tiling_on_sc=False, needs_layout_passes=False,
            disable_bounds_checks=True),
    )


def kernel(observed_pinn_in, observed_values):
    pinn8 = jnp.pad(observed_pinn_in[:_NFULL], ((0, 0), (0, 4)))
    pinn_s = pinn8.reshape(
        _NSTR, _STRIPE, 8).transpose(0, 2, 1).reshape(-1)
    vals_s = observed_values[:_NFULL].reshape(
        _NSTR, _STRIPE, 8).transpose(0, 2, 1).reshape(-1)
    tailp = observed_pinn_in[_NFULL:].reshape(-1)
    tailv = observed_values[_NFULL:].reshape(-1)
    out_p, out_v = _sc_gather()(
        pinn_s, vals_s, tailp, tailv,
        jnp.asarray(_PPOS), jnp.asarray(_VPOS), jnp.asarray(_FIX))
    pinn_b = out_p.reshape(_BS // _STRIPE, 4, _STRIPE).transpose(
        0, 2, 1).reshape(_BS, 4)
    vals_b = out_v.reshape(_BS // _STRIPE, 8, _STRIPE).transpose(
        0, 2, 1).reshape(_BS, 8)
    return (pinn_b, vals_b)
